# baseline (speedup 1.0000x reference)
import functools

import jax
import jax.numpy as jnp
from jax.experimental import pallas as pl
from jax.experimental.pallas import tpu as pltpu

_GAMMA = 0.2
_FW = 1.0 - _GAMMA

_NC = 5
_CH = 1000


def _fused_mlp_kernel(f_hbm, s_hbm, w1_ref, b1_ref, w2_ref, b2_ref, w3_ref,
                      b3_ref, out_hbm, fbuf, sbuf, obuf, fsem, ssem, osem):
    def f_copy(c):
        rows = pl.ds(c * _CH, _CH)
        return pltpu.make_async_copy(f_hbm.at[rows, :], fbuf.at[rows, :],
                                     fsem.at[c])

    def s_copy(c):
        rows = pl.ds(c * _CH, _CH)
        return pltpu.make_async_copy(s_hbm.at[rows, :], sbuf.at[rows, :],
                                     ssem.at[c])

    def o_copy(c):
        rows = pl.ds(c * _CH, _CH)
        return pltpu.make_async_copy(obuf.at[rows, :], out_hbm.at[rows, :],
                                     osem.at[c])

    for c in range(_NC):
        f_copy(c).start()
        s_copy(c).start()

    din = w1_ref.shape[0]
    h = w1_ref.shape[1]
    cdim = w3_ref.shape[1]
    zdh = jnp.zeros((din, h), jnp.float32)
    zhc = jnp.zeros((h, cdim), jnp.float32)
    w1p = jnp.concatenate(
        (jnp.concatenate((_FW * w1_ref[...], zdh), axis=1),
         jnp.concatenate((zdh, _FW * w1_ref[...]), axis=1)), axis=0
    ).astype(jnp.bfloat16)
    w2p = jnp.concatenate(
        (jnp.concatenate((_GAMMA * w2_ref[...], zdh), axis=1),
         jnp.concatenate((zdh, _GAMMA * w2_ref[...]), axis=1)), axis=0
    ).astype(jnp.bfloat16)
    w3p = jnp.concatenate(
        (jnp.concatenate((w3_ref[...], zhc), axis=1),
         jnp.concatenate((zhc, w3_ref[...]), axis=1)), axis=0
    ).astype(jnp.bfloat16)
    b12 = _FW * b1_ref[...] + _GAMMA * b2_ref[...]
    b12p = jnp.concatenate((b12, b12), axis=1)
    b3p = jnp.concatenate((b3_ref[...], b3_ref[...]), axis=1)

    for c in range(_NC):
        f_copy(c).wait()
        s_copy(c).wait()
        rows = pl.ds(c * _CH, _CH)
        ebd = jnp.dot(fbuf[rows, :].astype(jnp.bfloat16), w1p,
                      preferred_element_type=jnp.float32)
        ebd += jnp.dot(sbuf[rows, :].astype(jnp.bfloat16), w2p,
                       preferred_element_type=jnp.float32)
        ebd = jnp.maximum(ebd + b12p, 0.0)
        obuf[rows, :] = (
            jnp.dot(ebd.astype(jnp.bfloat16), w3p,
                    preferred_element_type=jnp.float32)
            + b3p
        )
        o_copy(c).start()

    for c in range(_NC):
        o_copy(c).wait()


@functools.partial(jax.jit, static_argnames=())
def _run(f, s, W1, b1, W2, b2, W3, b3):
    n, din = f.shape
    din3 = s.shape[1]
    h = W1.shape[1]
    c = W3.shape[1]
    np_ = n // 2

    f2 = f.reshape(np_, 2 * din)
    s2 = s.reshape(np_, 2 * din3)

    vmem = pltpu.MemorySpace.VMEM
    out2 = pl.pallas_call(
        _fused_mlp_kernel,
        in_specs=[
            pl.BlockSpec(memory_space=pl.ANY),
            pl.BlockSpec(memory_space=pl.ANY),
            pl.BlockSpec(memory_space=vmem),
            pl.BlockSpec(memory_space=vmem),
            pl.BlockSpec(memory_space=vmem),
            pl.BlockSpec(memory_space=vmem),
            pl.BlockSpec(memory_space=vmem),
            pl.BlockSpec(memory_space=vmem),
        ],
        out_specs=pl.BlockSpec(memory_space=pl.ANY),
        out_shape=jax.ShapeDtypeStruct((np_, 2 * c), jnp.float32),
        scratch_shapes=[
            vmem((np_, 2 * din), jnp.float32),
            vmem((np_, 2 * din3), jnp.float32),
            vmem((np_, 2 * c), jnp.float32),
            pltpu.SemaphoreType.DMA((_NC,)),
            pltpu.SemaphoreType.DMA((_NC,)),
            pltpu.SemaphoreType.DMA((_NC,)),
        ],
    )(f2, s2, W1, b1.reshape(1, h), W2, b2.reshape(1, h), W3,
      b3.reshape(1, c))
    return out2.reshape(n, c)


def kernel(f, s, adj_t, W1, b1, W2, b2, W3, b3):
    del adj_t
    return _run(f, s, W1, b1, W2, b2, W3, b3)


# speedup vs baseline: 1.5515x; 1.5515x over previous
import functools

import jax
import jax.numpy as jnp
from jax.experimental import pallas as pl
from jax.experimental.pallas import tpu as pltpu

_GAMMA = 0.2
_FW = 1.0 - _GAMMA

_NC = 10
_CH = 1000


def _fused_mlp_kernel(f_hbm, s_hbm, w1_ref, b1_ref, w2_ref, b2_ref, w3_ref,
                      b3_ref, out_hbm, fbuf, sbuf, obuf, fsem, ssem, osem):
    def f_copy(c):
        rows = pl.ds(c * _CH, _CH)
        return pltpu.make_async_copy(f_hbm.at[rows, :], fbuf.at[rows, :],
                                     fsem.at[c])

    def s_copy(c):
        rows = pl.ds(c * _CH, _CH)
        return pltpu.make_async_copy(s_hbm.at[rows, :], sbuf.at[rows, :],
                                     ssem.at[c])

    for c in range(_NC):
        f_copy(c).start()
        s_copy(c).start()

    w12 = jnp.concatenate(
        (_FW * w1_ref[...], _GAMMA * w2_ref[...]), axis=0
    ).astype(jnp.bfloat16)
    w3b = w3_ref[...].astype(jnp.bfloat16)
    b12 = _FW * b1_ref[...] + _GAMMA * b2_ref[...]
    b3v = b3_ref[...]

    for c in range(_NC):
        f_copy(c).wait()
        s_copy(c).wait()

    fs = jnp.concatenate((fbuf[...], sbuf[...]), axis=1)
    ebd = jnp.dot(fs.astype(jnp.bfloat16), w12,
                  preferred_element_type=jnp.float32)
    ebd = jnp.maximum(ebd + b12, 0.0)
    obuf[...] = (
        jnp.dot(ebd.astype(jnp.bfloat16), w3b,
                preferred_element_type=jnp.float32)
        + b3v
    )
    pltpu.make_async_copy(obuf, out_hbm, osem).start()
    pltpu.make_async_copy(obuf, out_hbm, osem).wait()


@functools.partial(jax.jit, static_argnames=())
def _run(f, s, W1, b1, W2, b2, W3, b3):
    n, din = f.shape
    din3 = s.shape[1]
    h = W1.shape[1]
    c = W3.shape[1]

    vmem = pltpu.MemorySpace.VMEM
    return pl.pallas_call(
        _fused_mlp_kernel,
        in_specs=[
            pl.BlockSpec(memory_space=pl.ANY),
            pl.BlockSpec(memory_space=pl.ANY),
            pl.BlockSpec(memory_space=vmem),
            pl.BlockSpec(memory_space=vmem),
            pl.BlockSpec(memory_space=vmem),
            pl.BlockSpec(memory_space=vmem),
            pl.BlockSpec(memory_space=vmem),
            pl.BlockSpec(memory_space=vmem),
        ],
        out_specs=pl.BlockSpec(memory_space=pl.ANY),
        out_shape=jax.ShapeDtypeStruct((n, c), jnp.float32),
        scratch_shapes=[
            vmem((n, din), jnp.float32),
            vmem((n, din3), jnp.float32),
            vmem((n, c), jnp.float32),
            pltpu.SemaphoreType.DMA((_NC,)),
            pltpu.SemaphoreType.DMA((_NC,)),
            pltpu.SemaphoreType.DMA(()),
        ],
    )(f, s, W1, b1.reshape(1, h), W2, b2.reshape(1, h), W3, b3.reshape(1, c))


def kernel(f, s, adj_t, W1, b1, W2, b2, W3, b3):
    del adj_t
    return _run(f, s, W1, b1, W2, b2, W3, b3)


# speedup vs baseline: 1.6141x; 1.0404x over previous
import functools

import jax
import jax.numpy as jnp
from jax.experimental import pallas as pl
from jax.experimental.pallas import tpu as pltpu

_GAMMA = 0.2
_FW = 1.0 - _GAMMA

_NC = 10
_CH = 1000


def _fused_mlp_kernel(f_hbm, s_hbm, w1_ref, b1_ref, w2_ref, b2_ref, w3_ref,
                      b3_ref, out_hbm, fbuf, sbuf, ebuf, obuf,
                      fsem, ssem, osem):
    def f_copy(c):
        rows = pl.ds(c * _CH, _CH)
        return pltpu.make_async_copy(f_hbm.at[rows, :], fbuf.at[rows, :],
                                     fsem.at[c])

    def s_copy(c):
        rows = pl.ds(c * _CH, _CH)
        return pltpu.make_async_copy(s_hbm.at[rows, :], sbuf.at[rows, :],
                                     ssem.at[c])

    def o_copy(c):
        rows = pl.ds(c * _CH, _CH)
        return pltpu.make_async_copy(obuf.at[rows, :], out_hbm.at[rows, :],
                                     osem.at[c])

    for c in range(_NC):
        f_copy(c).start()
        s_copy(c).start()

    w12 = jnp.concatenate(
        (_FW * w1_ref[...], _GAMMA * w2_ref[...]), axis=0
    ).astype(jnp.bfloat16)
    w3b = w3_ref[...].astype(jnp.bfloat16)
    b12 = _FW * b1_ref[...] + _GAMMA * b2_ref[...]
    b3v = b3_ref[...]

    for c in range(_NC):
        f_copy(c).wait()
        s_copy(c).wait()
        rows = pl.ds(c * _CH, _CH)
        fs = jnp.concatenate((fbuf[rows, :], sbuf[rows, :]), axis=1)
        ebuf[rows, :] = jnp.dot(fs.astype(jnp.bfloat16), w12,
                                preferred_element_type=jnp.float32)

    for c in range(_NC):
        rows = pl.ds(c * _CH, _CH)
        ebd = jnp.maximum(ebuf[rows, :] + b12, 0.0)
        obuf[rows, :] = (
            jnp.dot(ebd.astype(jnp.bfloat16), w3b,
                    preferred_element_type=jnp.float32)
            + b3v
        )
        o_copy(c).start()

    for c in range(_NC):
        o_copy(c).wait()


@functools.partial(jax.jit, static_argnames=())
def _run(f, s, W1, b1, W2, b2, W3, b3):
    n, din = f.shape
    din3 = s.shape[1]
    h = W1.shape[1]
    c = W3.shape[1]

    vmem = pltpu.MemorySpace.VMEM
    return pl.pallas_call(
        _fused_mlp_kernel,
        in_specs=[
            pl.BlockSpec(memory_space=pl.ANY),
            pl.BlockSpec(memory_space=pl.ANY),
            pl.BlockSpec(memory_space=vmem),
            pl.BlockSpec(memory_space=vmem),
            pl.BlockSpec(memory_space=vmem),
            pl.BlockSpec(memory_space=vmem),
            pl.BlockSpec(memory_space=vmem),
            pl.BlockSpec(memory_space=vmem),
        ],
        out_specs=pl.BlockSpec(memory_space=pl.ANY),
        out_shape=jax.ShapeDtypeStruct((n, c), jnp.float32),
        scratch_shapes=[
            vmem((n, din), jnp.float32),
            vmem((n, din3), jnp.float32),
            vmem((n, h), jnp.float32),
            vmem((n, c), jnp.float32),
            pltpu.SemaphoreType.DMA((_NC,)),
            pltpu.SemaphoreType.DMA((_NC,)),
            pltpu.SemaphoreType.DMA((_NC,)),
        ],
    )(f, s, W1, b1.reshape(1, h), W2, b2.reshape(1, h), W3, b3.reshape(1, c))


def kernel(f, s, adj_t, W1, b1, W2, b2, W3, b3):
    del adj_t
    return _run(f, s, W1, b1, W2, b2, W3, b3)


# speedup vs baseline: 1.6776x; 1.0393x over previous
import functools

import jax
import jax.numpy as jnp
from jax.experimental import pallas as pl
from jax.experimental.pallas import tpu as pltpu

_GAMMA = 0.2
_FW = 1.0 - _GAMMA

_NC = 5
_CH = 2000


def _fused_mlp_kernel(f_hbm, s_hbm, w1_ref, b1_ref, w2_ref, b2_ref, w3_ref,
                      b3_ref, out_hbm, fbuf, sbuf, ebuf, obuf,
                      fsem, ssem, osem):
    def f_copy(c):
        rows = pl.ds(c * _CH, _CH)
        return pltpu.make_async_copy(f_hbm.at[rows, :], fbuf.at[rows, :],
                                     fsem.at[c])

    def s_copy(c):
        rows = pl.ds(c * _CH, _CH)
        return pltpu.make_async_copy(s_hbm.at[rows, :], sbuf.at[rows, :],
                                     ssem.at[c])

    def o_copy(c):
        rows = pl.ds(c * _CH, _CH)
        return pltpu.make_async_copy(obuf.at[rows, :], out_hbm.at[rows, :],
                                     osem.at[c])

    for c in range(_NC):
        f_copy(c).start()
        s_copy(c).start()

    w12 = jnp.concatenate(
        (_FW * w1_ref[...], _GAMMA * w2_ref[...]), axis=0
    ).astype(jnp.bfloat16)
    w3b = w3_ref[...].astype(jnp.bfloat16)
    b12 = _FW * b1_ref[...] + _GAMMA * b2_ref[...]
    b3v = b3_ref[...]

    for c in range(_NC):
        f_copy(c).wait()
        s_copy(c).wait()
        rows = pl.ds(c * _CH, _CH)
        fs = jnp.concatenate((fbuf[rows, :], sbuf[rows, :]), axis=1)
        ebd = jnp.dot(fs.astype(jnp.bfloat16), w12,
                      preferred_element_type=jnp.float32)
        ebuf[rows, :] = jnp.maximum(ebd + b12, 0.0).astype(jnp.bfloat16)

    for c in range(_NC):
        rows = pl.ds(c * _CH, _CH)
        obuf[rows, :] = (
            jnp.dot(ebuf[rows, :], w3b, preferred_element_type=jnp.float32)
            + b3v
        )
        o_copy(c).start()

    for c in range(_NC):
        o_copy(c).wait()


@functools.partial(jax.jit, static_argnames=())
def _run(f, s, W1, b1, W2, b2, W3, b3):
    n, din = f.shape
    din3 = s.shape[1]
    h = W1.shape[1]
    c = W3.shape[1]

    vmem = pltpu.MemorySpace.VMEM
    return pl.pallas_call(
        _fused_mlp_kernel,
        in_specs=[
            pl.BlockSpec(memory_space=pl.ANY),
            pl.BlockSpec(memory_space=pl.ANY),
            pl.BlockSpec(memory_space=vmem),
            pl.BlockSpec(memory_space=vmem),
            pl.BlockSpec(memory_space=vmem),
            pl.BlockSpec(memory_space=vmem),
            pl.BlockSpec(memory_space=vmem),
            pl.BlockSpec(memory_space=vmem),
        ],
        out_specs=pl.BlockSpec(memory_space=pl.ANY),
        out_shape=jax.ShapeDtypeStruct((n, c), jnp.float32),
        scratch_shapes=[
            vmem((n, din), jnp.float32),
            vmem((n, din3), jnp.float32),
            vmem((n, h), jnp.bfloat16),
            vmem((n, c), jnp.float32),
            pltpu.SemaphoreType.DMA((_NC,)),
            pltpu.SemaphoreType.DMA((_NC,)),
            pltpu.SemaphoreType.DMA((_NC,)),
        ],
    )(f, s, W1, b1.reshape(1, h), W2, b2.reshape(1, h), W3, b3.reshape(1, c))


def kernel(f, s, adj_t, W1, b1, W2, b2, W3, b3):
    del adj_t
    return _run(f, s, W1, b1, W2, b2, W3, b3)


# speedup vs baseline: 1.7660x; 1.0527x over previous
import functools

import jax
import jax.numpy as jnp
from jax.experimental import pallas as pl
from jax.experimental.pallas import tpu as pltpu

_GAMMA = 0.2
_FW = 1.0 - _GAMMA

_NC = 5
_CH = 2000


def _fused_mlp_kernel(f_hbm, s_hbm, w1_ref, b1_ref, w2_ref, b2_ref, w3_ref,
                      b3_ref, out_hbm, fbuf, sbuf, obuf,
                      fsem, ssem, osem):
    def f_copy(c):
        rows = pl.ds(c * _CH, _CH)
        return pltpu.make_async_copy(f_hbm.at[rows, :], fbuf.at[rows, :],
                                     fsem.at[c])

    def s_copy(c):
        rows = pl.ds(c * _CH, _CH)
        return pltpu.make_async_copy(s_hbm.at[rows, :], sbuf.at[rows, :],
                                     ssem.at[c])

    def o_copy(c):
        rows = pl.ds(c * _CH, _CH)
        return pltpu.make_async_copy(obuf.at[rows, :], out_hbm.at[rows, :],
                                     osem.at[c])

    for c in range(_NC):
        f_copy(c).start()
        s_copy(c).start()

    w12 = jnp.concatenate(
        (_FW * w1_ref[...], _GAMMA * w2_ref[...]), axis=0
    ).astype(jnp.bfloat16)
    w3b = w3_ref[...].astype(jnp.bfloat16)
    b12 = _FW * b1_ref[...] + _GAMMA * b2_ref[...]
    b3v = b3_ref[...]

    for c in range(_NC):
        f_copy(c).wait()
        s_copy(c).wait()
        rows = pl.ds(c * _CH, _CH)
        fs = jnp.concatenate((fbuf[rows, :], sbuf[rows, :]), axis=1)
        ebd = jnp.dot(fs.astype(jnp.bfloat16), w12,
                      preferred_element_type=jnp.float32)
        ebd = jnp.maximum(ebd + b12, 0.0).astype(jnp.bfloat16)
        obuf[rows, :] = (
            jnp.dot(ebd, w3b, preferred_element_type=jnp.float32)
            + b3v
        )
        o_copy(c).start()

    for c in range(_NC):
        o_copy(c).wait()


@functools.partial(jax.jit, static_argnames=())
def _run(f, s, W1, b1, W2, b2, W3, b3):
    n, din = f.shape
    din3 = s.shape[1]
    h = W1.shape[1]
    c = W3.shape[1]

    vmem = pltpu.MemorySpace.VMEM
    return pl.pallas_call(
        _fused_mlp_kernel,
        in_specs=[
            pl.BlockSpec(memory_space=pl.ANY),
            pl.BlockSpec(memory_space=pl.ANY),
            pl.BlockSpec(memory_space=vmem),
            pl.BlockSpec(memory_space=vmem),
            pl.BlockSpec(memory_space=vmem),
            pl.BlockSpec(memory_space=vmem),
            pl.BlockSpec(memory_space=vmem),
            pl.BlockSpec(memory_space=vmem),
        ],
        out_specs=pl.BlockSpec(memory_space=pl.ANY),
        out_shape=jax.ShapeDtypeStruct((n, c), jnp.float32),
        scratch_shapes=[
            vmem((n, din), jnp.float32),
            vmem((n, din3), jnp.float32),
            vmem((n, c), jnp.float32),
            pltpu.SemaphoreType.DMA((_NC,)),
            pltpu.SemaphoreType.DMA((_NC,)),
            pltpu.SemaphoreType.DMA((_NC,)),
        ],
    )(f, s, W1, b1.reshape(1, h), W2, b2.reshape(1, h), W3, b3.reshape(1, c))


def kernel(f, s, adj_t, W1, b1, W2, b2, W3, b3):
    del adj_t
    return _run(f, s, W1, b1, W2, b2, W3, b3)
